# baseline (speedup 1.0000x reference)
import functools

import jax
import jax.numpy as jnp
from jax import lax
from jax.experimental import pallas as pl
from jax.experimental.pallas import tpu as pltpu, tpu_sc as plsc

_RC = 128


def _make_gather(B: int, D: int):
    info = plsc.get_sparse_core_info()
    NC, NS = info.num_cores, info.num_subcores
    NW = NC * NS
    assert B % (8 * NW) == 0
    b_per_w = B // NW
    n_chunks = b_per_w // _RC
    assert n_chunks * _RC == b_per_w
    mesh = plsc.VectorSubcoreMesh(core_axis_name="c", subcore_axis_name="s")

    @functools.partial(
        pl.kernel,
        mesh=mesh,
        out_type=jax.ShapeDtypeStruct((B, D), jnp.float32),
        compiler_params=pltpu.CompilerParams(needs_layout_passes=False),
        scratch_types=[
            pltpu.VMEM((b_per_w,), jnp.int32),
            pltpu.VMEM((_RC, D), jnp.float32),
            pltpu.SemaphoreType.DMA,
        ],
    )
    def gather_kernel(idx_hbm, table_hbm, out_hbm, idx_s, rows_v, sem):
        wid = lax.axis_index("s") * NC + lax.axis_index("c")
        base = wid * b_per_w
        pltpu.sync_copy(idx_hbm.at[pl.ds(base, b_per_w)], idx_s)

        def chunk_body(k):
            def fire(g):
                v = idx_s[pl.ds(k * _RC + g * 16, 16)]
                for l in range(16):
                    pltpu.async_copy(table_hbm.at[v[l]], rows_v.at[g * 16 + l], sem)

            pl.loop(0, _RC // 16)(fire)
            pltpu.make_async_copy(out_hbm.at[pl.ds(0, _RC)], rows_v, sem).wait()
            pltpu.sync_copy(rows_v, out_hbm.at[pl.ds(base + k * _RC, _RC)])

        pl.loop(0, n_chunks)(chunk_body)

    return gather_kernel


def kernel(batch_ids, latents):
    B = batch_ids.shape[0]
    D = latents.shape[1]
    idx = batch_ids.astype(jnp.int32)
    out = _make_gather(B, D)(idx, latents)
    return out.reshape(B, 1, 1, D)
